# baseline (speedup 1.0000x reference)
import jax
import jax.numpy as jnp
from jax.experimental import pallas as pl
from jax.experimental.pallas import tpu as pltpu
from jax.experimental.pallas import tpu_sc as plsc

_BS = 512
_BR = 8
_LANES = 16
_Q = 2560
_JBS = 512


def _add_kernel(x_ref, pe_ref, o_ref):
    o_ref[...] = x_ref[...] + pe_ref[...][None, :, :]


def _tc_part(x, pe_table, n_rows):
    B, S, D = x.shape
    return pl.pallas_call(
        _add_kernel,
        grid=(n_rows // _BS,),
        in_specs=[
            pl.BlockSpec((B, _BS, D), lambda i: (0, i, 0)),
            pl.BlockSpec((_BS, D), lambda i: (i, 0)),
        ],
        out_specs=pl.BlockSpec((B, _BS, D), lambda i: (0, i, 0)),
        out_shape=jax.ShapeDtypeStruct((B, S, D), x.dtype),
        compiler_params=pltpu.CompilerParams(dimension_semantics=("parallel",)),
    )(x, pe_table)


def _sc_part(x, pe_table, q):
    B, S, D = x.shape
    x2 = x.reshape(B * S, D)
    sb_total = S // _BR
    qb = q // _BR
    base = (S - q) // _BR
    mesh = plsc.VectorSubcoreMesh(core_axis_name="core", subcore_axis_name="subcore")

    def _x_map(b):
        return lambda i: (b * sb_total + base + i, 0)

    def _o_map(b):
        return lambda i: (b * qb + i, 0)

    @pl.kernel(out_type=jax.ShapeDtypeStruct((B * q, D), x.dtype), mesh=mesh)
    def sc_kern(x_hbm, pe_hbm, o_hbm):
        def body(*refs):
            x_refs = refs[:B]
            pe_vmem = refs[B]
            o_refs = refs[B + 1:]

            @pl.loop(0, _BR)
            def _row(r):
                @plsc.parallel_loop(0, D, step=_LANES, unroll=4)
                def _col(c):
                    slc = (pl.ds(r, 1), pl.ds(c, _LANES))
                    pv = pe_vmem.at[slc][...]
                    for xb, ob in zip(x_refs, o_refs):
                        ob.at[slc][...] = xb.at[slc][...] + pv

        pltpu.emit_pipeline(
            body,
            grid=(qb,),
            in_specs=[pl.BlockSpec((_BR, D), index_map=_x_map(b)) for b in range(B)]
            + [pl.BlockSpec((_BR, D), index_map=lambda i: (base + i, 0))],
            out_specs=[pl.BlockSpec((_BR, D), index_map=_o_map(b)) for b in range(B)],
            core_axis_name=("core", "subcore"),
            dimension_semantics=(pltpu.PARALLEL,),
        )(*([x_hbm] * B), pe_hbm, *([o_hbm] * B))

    return sc_kern(x2, pe_table).reshape(B, q, D)


def _join_kernel(full_ref, sc_ref, o_ref):
    o_ref[...] = sc_ref[...]


def _join(tc_full, sc_tail, q):
    B, S, D = tc_full.shape
    base = (S - q) // _JBS
    return pl.pallas_call(
        _join_kernel,
        grid=(q // _JBS, B),
        in_specs=[
            pl.BlockSpec(memory_space=pl.ANY),
            pl.BlockSpec((1, _JBS, D), lambda i, b: (b, i, 0)),
        ],
        out_specs=pl.BlockSpec((1, _JBS, D), lambda i, b: (b, base + i, 0)),
        out_shape=jax.ShapeDtypeStruct((B, S, D), tc_full.dtype),
        input_output_aliases={0: 0},
        compiler_params=pltpu.CompilerParams(
            dimension_semantics=("parallel", "parallel")
        ),
    )(tc_full, sc_tail)


def kernel(x, pe_table):
    B, S, D = x.shape
    tc_full = _tc_part(x, pe_table, S - _Q)
    sc_tail = _sc_part(x, pe_table, _Q)
    return _join(tc_full, sc_tail, _Q)


# speedup vs baseline: 1.1403x; 1.1403x over previous
import jax
import jax.numpy as jnp
from jax.experimental import pallas as pl
from jax.experimental.pallas import tpu as pltpu
from jax.experimental.pallas import tpu_sc as plsc

_BR = 16
_LANES = 16
_PAIR = 2


def kernel(x, pe_table):
    B, S, D = x.shape
    SB = S // _BR
    n_pairs = B // _PAIR
    x2 = x.reshape(B * S, D)
    mesh = plsc.VectorSubcoreMesh(core_axis_name="core", subcore_axis_name="subcore")

    def _x_map(j):
        return lambda i, g: ((_PAIR * g + j) * SB + i, 0)

    @pl.kernel(out_type=jax.ShapeDtypeStruct((B * S, D), x.dtype), mesh=mesh)
    def sc_kern(x_hbm, pe_hbm, o_hbm):
        def body(*refs):
            x_refs = refs[:_PAIR]
            pe_vmem = refs[_PAIR]
            o_refs = refs[_PAIR + 1:]

            @pl.loop(0, _BR)
            def _row(r):
                @plsc.parallel_loop(0, D, step=_LANES, unroll=4)
                def _col(c):
                    slc = (pl.ds(r, 1), pl.ds(c, _LANES))
                    pv = pe_vmem.at[slc][...]
                    for xb, ob in zip(x_refs, o_refs):
                        ob.at[slc][...] = xb.at[slc][...] + pv

        pltpu.emit_pipeline(
            body,
            grid=(SB, n_pairs),
            in_specs=[pl.BlockSpec((_BR, D), index_map=_x_map(j)) for j in range(_PAIR)]
            + [pl.BlockSpec((_BR, D), index_map=lambda i, g: (i, 0))],
            out_specs=[pl.BlockSpec((_BR, D), index_map=_x_map(j)) for j in range(_PAIR)],
            core_axis_name=("core", "subcore"),
            dimension_semantics=(pltpu.PARALLEL, pltpu.ARBITRARY),
        )(*([x_hbm] * _PAIR), pe_hbm, *([o_hbm] * _PAIR))

    return sc_kern(x2, pe_table).reshape(B, S, D)
